# baseline (speedup 1.0000x reference)
import functools

import jax
import jax.numpy as jnp
from jax import lax
from jax.experimental import pallas as pl
from jax.experimental.pallas import tpu as pltpu
from jax.experimental.pallas import tpu_sc as plsc

NC = 2
NS = 16
NW = NC * NS
CH = 128
SB = 8
LANES = 16
T0 = 40
T1 = 120


@functools.lru_cache(maxsize=None)
def _make_sc_aggregate(n, d):
    rows_per_tile = (((n + NS - 1) // NS + 127) // 128) * 128
    n_pad = NS * rows_per_tile
    zrows = CH
    nzc = rows_per_tile // zrows
    fgroups = d // LANES
    assert T0 % SB == 0 and T1 % SB == 0
    nsb0 = T0 // SB
    nsb1 = T1 // SB

    mesh = plsc.VectorSubcoreMesh(core_axis_name="c", subcore_axis_name="s")

    @functools.partial(
        pl.kernel,
        mesh=mesh,
        out_type=jax.ShapeDtypeStruct((2 * n_pad, d), jnp.float32),
        scratch_types=[
            pltpu.VMEM((SB, CH), jnp.int32),
            pltpu.VMEM((SB, CH), jnp.int32),
            pltpu.VMEM((SB, CH), jnp.float32),
            pltpu.VMEM((CH, d), jnp.float32),
            pltpu.VMEM((CH, d), jnp.float32),
            pltpu.VMEM_SHARED((n_pad, d), jnp.float32),
            pltpu.SemaphoreType.DMA,
            pltpu.SemaphoreType.DMA,
            pltpu.SemaphoreType.DMA,
            pltpu.SemaphoreType.DMA,
        ],
    )
    def agg_kernel(src_hbm, dst_hbm, w_hbm, x_hbm, out_hbm,
                   src_v, dst_v, w_v, rows0, rows1, acc_sh,
                   gs0, gs1, ss0, ss1):
        cid = lax.axis_index("c")
        sid = lax.axis_index("s")
        base_chunk = jnp.where(cid == 0, sid * T0, NS * T0 + sid * T1)
        nsb = jnp.where(cid == 0, nsb0, nsb1)

        def zrow(i, carry):
            for f in range(fgroups):
                rows0[i, pl.ds(f * LANES, LANES)] = jnp.zeros(
                    (LANES,), jnp.float32)
            return carry
        lax.fori_loop(0, zrows, zrow, 0)
        for k in range(nzc):
            pltpu.sync_copy(
                rows0,
                acc_sh.at[pl.ds(sid * rows_per_tile + k * zrows, zrows)])
        plsc.subcore_barrier()

        def scale(buf, j):
            def egroup(g, c2):
                wv16 = w_v[j, pl.ds(g * LANES, LANES)]
                for l in range(LANES):
                    wb = jnp.full((LANES,), wv16[l], dtype=jnp.float32)
                    i = g * LANES + l
                    for f in range(fgroups):
                        sl = pl.ds(f * LANES, LANES)
                        buf[i, sl] = buf[i, sl] * wb
                return c2
            lax.fori_loop(0, CH // LANES, egroup, 0)

        def start_gather(buf, sem, j):
            pltpu.async_copy(x_hbm.at[src_v.at[j]], buf, sem)

        def wait_gather(buf, sem):
            pltpu.make_async_copy(x_hbm.at[src_v.at[0]], buf, sem).wait()

        def start_scatter(buf, sem, j):
            pltpu.async_copy(buf, acc_sh.at[dst_v.at[j]], sem, add=True)

        def wait_scatter(buf, sem):
            pltpu.make_async_copy(buf, acc_sh.at[dst_v.at[0]], sem).wait()

        def block(s, carry):
            cb = base_chunk + s * SB
            pltpu.sync_copy(src_hbm.at[pl.ds(cb, SB)], src_v)
            pltpu.sync_copy(dst_hbm.at[pl.ds(cb, SB)], dst_v)
            pltpu.sync_copy(w_hbm.at[pl.ds(cb, SB)], w_v)

            start_gather(rows0, gs0, 0)
            wait_gather(rows0, gs0)
            scale(rows0, 0)
            start_gather(rows1, gs1, 1)
            start_scatter(rows0, ss0, 0)

            def pair(j2, c1):
                j = 2 * j2 + 1
                wait_gather(rows1, gs1)
                scale(rows1, j)
                wait_scatter(rows0, ss0)
                start_gather(rows0, gs0, j + 1)
                start_scatter(rows1, ss1, j)
                wait_gather(rows0, gs0)
                scale(rows0, j + 1)
                wait_scatter(rows1, ss1)
                start_gather(rows1, gs1, j + 2)
                start_scatter(rows0, ss0, j + 1)
                return c1
            lax.fori_loop(0, (SB - 2) // 2, pair, 0)

            wait_gather(rows1, gs1)
            scale(rows1, SB - 1)
            wait_scatter(rows0, ss0)
            start_scatter(rows1, ss1, SB - 1)
            wait_scatter(rows1, ss1)
            return carry
        lax.fori_loop(0, nsb, block, 0)

        plsc.subcore_barrier()
        pltpu.sync_copy(
            acc_sh.at[pl.ds(sid * rows_per_tile, rows_per_tile)],
            out_hbm.at[pl.ds(cid * n_pad + sid * rows_per_tile,
                             rows_per_tile)])

    return agg_kernel


def _mlp0_body(x_ref, aggA_ref, aggB_ref, scale_ref, w1_ref, b1_ref,
               w2_ref, b2_ref, h0_ref, a0_ref):
    h = scale_ref[0, 0] * x_ref[...] + aggA_ref[...] + aggB_ref[...]
    t = jnp.dot(h, w1_ref[...], preferred_element_type=jnp.float32)
    t = t + b1_ref[...]
    t = jnp.where(t > 0, t, 0.01 * t)
    h0 = jnp.dot(t, w2_ref[...], preferred_element_type=jnp.float32)
    h0 = h0 + b2_ref[...]
    h0_ref[...] = h0
    a0_ref[...] = jnp.where(h0 > 0, h0, 0.01 * h0)


@functools.lru_cache(maxsize=None)
def _make_mlp0(n, d, blk):
    nb = n // blk
    wspec = pl.BlockSpec((d, d), lambda i: (0, 0))
    bspec = pl.BlockSpec((1, d), lambda i: (0, 0))
    return pl.pallas_call(
        _mlp0_body,
        grid=(nb,),
        in_specs=[
            pl.BlockSpec((blk, d), lambda i: (i, 0)),
            pl.BlockSpec((blk, d), lambda i: (i, 0)),
            pl.BlockSpec((blk, d), lambda i: (i, 0)),
            pl.BlockSpec(memory_space=pltpu.SMEM),
            wspec, bspec, wspec, bspec,
        ],
        out_specs=[
            pl.BlockSpec((blk, d), lambda i: (i, 0)),
            pl.BlockSpec((blk, d), lambda i: (i, 0)),
        ],
        out_shape=[
            jax.ShapeDtypeStruct((n, d), jnp.float32),
            jax.ShapeDtypeStruct((n, d), jnp.float32),
        ],
    )


@functools.lru_cache(maxsize=None)
def _make_mlp1_pool(n, d, b, blk):
    nb = n // blk

    def body(a0_ref, aggA_ref, aggB_ref, scale_ref, w1_ref, b1_ref,
             w2_ref, b2_ref, h0_ref, batch_ref, lw0_ref, lw1_ref, lb_ref,
             out_ref, p0_ref, p1_ref):
        i = pl.program_id(0)
        h = scale_ref[0, 0] * a0_ref[...] + aggA_ref[...] + aggB_ref[...]
        t = jnp.dot(h, w1_ref[...], preferred_element_type=jnp.float32)
        t = t + b1_ref[...]
        t = jnp.where(t > 0, t, 0.01 * t)
        h1 = jnp.dot(t, w2_ref[...], preferred_element_type=jnp.float32)
        h1 = h1 + b2_ref[...]

        seg = batch_ref[0, 0, :]
        mask = (seg[None, :] ==
                lax.broadcasted_iota(jnp.int32, (b, blk), 0)).astype(
                    jnp.float32)
        pool0 = jnp.dot(mask, h0_ref[...], preferred_element_type=jnp.float32)
        pool1 = jnp.dot(mask, h1, preferred_element_type=jnp.float32)

        @pl.when(i == 0)
        def _():
            p0_ref[...] = jnp.zeros((b, d), jnp.float32)
            p1_ref[...] = jnp.zeros((b, d), jnp.float32)

        p0_ref[...] += pool0
        p1_ref[...] += pool1

        @pl.when(i == nb - 1)
        def _():
            out_ref[...] = (
                jnp.dot(p0_ref[...], lw0_ref[...],
                        preferred_element_type=jnp.float32)
                + jnp.dot(p1_ref[...], lw1_ref[...],
                          preferred_element_type=jnp.float32)
                + lb_ref[...])

    wspec = pl.BlockSpec((d, d), lambda i: (0, 0))
    bspec = pl.BlockSpec((1, d), lambda i: (0, 0))
    return pl.pallas_call(
        body,
        grid=(nb,),
        in_specs=[
            pl.BlockSpec((blk, d), lambda i: (i, 0)),
            pl.BlockSpec((blk, d), lambda i: (i, 0)),
            pl.BlockSpec((blk, d), lambda i: (i, 0)),
            pl.BlockSpec(memory_space=pltpu.SMEM),
            wspec, bspec, wspec, bspec,
            pl.BlockSpec((blk, d), lambda i: (i, 0)),
            pl.BlockSpec((1, 1, blk), lambda i: (i, 0, 0)),
            wspec,
            pl.BlockSpec((d, d), lambda i: (0, 0)),
            bspec,
        ],
        out_specs=pl.BlockSpec((b, d), lambda i: (0, 0)),
        out_shape=jax.ShapeDtypeStruct((b, d), jnp.float32),
        scratch_shapes=[
            pltpu.VMEM((b, d), jnp.float32),
            pltpu.VMEM((b, d), jnp.float32),
        ],
    )


def kernel(x, edge_index, edge_weight, batch, gin0_w1, gin0_b1, gin0_w2,
           gin0_b2, eps0, gin1_w1, gin1_b1, gin1_w2, gin1_b2, eps1,
           lin_w, lin_b):
    n, d = x.shape
    e = edge_weight.shape[0]
    b = 64
    blk = 1000

    nchunks = NS * (T0 + T1)
    e_pad = nchunks * CH
    assert e_pad >= e
    pad = e_pad - e
    src_r = jnp.pad(edge_index[0], (0, pad)).reshape(nchunks, CH)
    dst_r = jnp.pad(edge_index[1], (0, pad)).reshape(nchunks, CH)
    w_r = jnp.pad(edge_weight, (0, pad)).reshape(nchunks, CH)

    sc_agg = _make_sc_aggregate(n, d)
    mlp0 = _make_mlp0(n, d, blk)
    mlp1_pool = _make_mlp1_pool(n, d, b, blk)

    scale0 = (1.0 + eps0).reshape(1, 1).astype(jnp.float32)
    scale1 = (1.0 + eps1).reshape(1, 1).astype(jnp.float32)
    b1_0 = gin0_b1.reshape(1, d)
    b2_0 = gin0_b2.reshape(1, d)
    b1_1 = gin1_b1.reshape(1, d)
    b2_1 = gin1_b2.reshape(1, d)
    lw0 = lin_w[:d]
    lw1 = lin_w[d:]
    lb = lin_b.reshape(1, d)
    batch3 = batch.reshape(n // blk, 1, blk)

    rpt = (((n + NS - 1) // NS + 127) // 128) * 128
    n_pad = NS * rpt

    agg0 = sc_agg(src_r, dst_r, w_r, x)
    h0, a0 = mlp0(x, agg0[:n], agg0[n_pad:n_pad + n], scale0,
                  gin0_w1, b1_0, gin0_w2, b2_0)
    agg1 = sc_agg(src_r, dst_r, w_r, a0)
    return mlp1_pool(a0, agg1[:n], agg1[n_pad:n_pad + n], scale1,
                     gin1_w1, b1_1, gin1_w2, b2_1,
                     h0, batch3, lw0, lw1, lb)


# speedup vs baseline: 1.3624x; 1.3624x over previous
import functools

import jax
import jax.numpy as jnp
from jax import lax
from jax.experimental import pallas as pl
from jax.experimental.pallas import tpu as pltpu
from jax.experimental.pallas import tpu_sc as plsc

NC = 2
NS = 16
NW = NC * NS
CH = 128
SB = 8
LANES = 16
T0 = 120
T1 = 40


@functools.lru_cache(maxsize=None)
def _make_sc_aggregate(n, d):
    rows_per_tile = (((n + NS - 1) // NS + 127) // 128) * 128
    n_pad = NS * rows_per_tile
    zrows = CH
    nzc = rows_per_tile // zrows
    fgroups = d // LANES
    assert T0 % SB == 0 and T1 % SB == 0
    nsb0 = T0 // SB
    nsb1 = T1 // SB

    mesh = plsc.VectorSubcoreMesh(core_axis_name="c", subcore_axis_name="s")

    @functools.partial(
        pl.kernel,
        mesh=mesh,
        out_type=jax.ShapeDtypeStruct((2 * n_pad, d), jnp.float32),
        scratch_types=[
            pltpu.VMEM((SB, CH), jnp.int32),
            pltpu.VMEM((SB, CH), jnp.int32),
            pltpu.VMEM((SB, CH), jnp.float32),
            pltpu.VMEM((CH, d), jnp.float32),
            pltpu.VMEM((CH, d), jnp.float32),
            pltpu.VMEM_SHARED((n_pad, d), jnp.float32),
            pltpu.SemaphoreType.DMA,
            pltpu.SemaphoreType.DMA,
            pltpu.SemaphoreType.DMA,
            pltpu.SemaphoreType.DMA,
        ],
    )
    def agg_kernel(src_hbm, dst_hbm, w_hbm, x_hbm, out_hbm,
                   src_v, dst_v, w_v, rows0, rows1, acc_sh,
                   gs0, gs1, ss0, ss1):
        cid = lax.axis_index("c")
        sid = lax.axis_index("s")
        base_chunk = jnp.where(cid == 0, sid * T0, NS * T0 + sid * T1)
        nsb = jnp.where(cid == 0, nsb0, nsb1)

        def zrow(i, carry):
            for f in range(fgroups):
                rows0[i, pl.ds(f * LANES, LANES)] = jnp.zeros(
                    (LANES,), jnp.float32)
            return carry
        lax.fori_loop(0, zrows, zrow, 0)
        for k in range(nzc):
            pltpu.sync_copy(
                rows0,
                acc_sh.at[pl.ds(sid * rows_per_tile + k * zrows, zrows)])
        plsc.subcore_barrier()

        def scale(buf, j):
            def egroup(g, c2):
                wv16 = w_v[j, pl.ds(g * LANES, LANES)]
                for l in range(LANES):
                    wb = jnp.full((LANES,), wv16[l], dtype=jnp.float32)
                    i = g * LANES + l
                    for f in range(fgroups):
                        sl = pl.ds(f * LANES, LANES)
                        buf[i, sl] = buf[i, sl] * wb
                return c2
            lax.fori_loop(0, CH // LANES, egroup, 0)

        def start_gather(buf, sem, j):
            pltpu.async_copy(x_hbm.at[src_v.at[j]], buf, sem)

        def wait_gather(buf, sem):
            pltpu.make_async_copy(x_hbm.at[src_v.at[0]], buf, sem).wait()

        def start_scatter(buf, sem, j):
            pltpu.async_copy(buf, acc_sh.at[dst_v.at[j]], sem, add=True)

        def wait_scatter(buf, sem):
            pltpu.make_async_copy(buf, acc_sh.at[dst_v.at[0]], sem).wait()

        def block(s, carry):
            cb = base_chunk + s * SB
            pltpu.sync_copy(src_hbm.at[pl.ds(cb, SB)], src_v)
            pltpu.sync_copy(dst_hbm.at[pl.ds(cb, SB)], dst_v)
            pltpu.sync_copy(w_hbm.at[pl.ds(cb, SB)], w_v)

            start_gather(rows0, gs0, 0)
            wait_gather(rows0, gs0)
            scale(rows0, 0)
            start_gather(rows1, gs1, 1)
            start_scatter(rows0, ss0, 0)

            def pair(j2, c1):
                j = 2 * j2 + 1
                wait_gather(rows1, gs1)
                scale(rows1, j)
                wait_scatter(rows0, ss0)
                start_gather(rows0, gs0, j + 1)
                start_scatter(rows1, ss1, j)
                wait_gather(rows0, gs0)
                scale(rows0, j + 1)
                wait_scatter(rows1, ss1)
                start_gather(rows1, gs1, j + 2)
                start_scatter(rows0, ss0, j + 1)
                return c1
            lax.fori_loop(0, (SB - 2) // 2, pair, 0)

            wait_gather(rows1, gs1)
            scale(rows1, SB - 1)
            wait_scatter(rows0, ss0)
            start_scatter(rows1, ss1, SB - 1)
            wait_scatter(rows1, ss1)
            return carry
        lax.fori_loop(0, nsb, block, 0)

        plsc.subcore_barrier()
        pltpu.sync_copy(
            acc_sh.at[pl.ds(sid * rows_per_tile, rows_per_tile)],
            out_hbm.at[pl.ds(cid * n_pad + sid * rows_per_tile,
                             rows_per_tile)])

    return agg_kernel


def _mlp0_body(x_ref, aggA_ref, aggB_ref, scale_ref, w1_ref, b1_ref,
               w2_ref, b2_ref, h0_ref, a0_ref):
    h = scale_ref[0, 0] * x_ref[...] + aggA_ref[...] + aggB_ref[...]
    t = jnp.dot(h, w1_ref[...], preferred_element_type=jnp.float32)
    t = t + b1_ref[...]
    t = jnp.where(t > 0, t, 0.01 * t)
    h0 = jnp.dot(t, w2_ref[...], preferred_element_type=jnp.float32)
    h0 = h0 + b2_ref[...]
    h0_ref[...] = h0
    a0_ref[...] = jnp.where(h0 > 0, h0, 0.01 * h0)


@functools.lru_cache(maxsize=None)
def _make_mlp0(n, d, blk):
    nb = n // blk
    wspec = pl.BlockSpec((d, d), lambda i: (0, 0))
    bspec = pl.BlockSpec((1, d), lambda i: (0, 0))
    return pl.pallas_call(
        _mlp0_body,
        grid=(nb,),
        in_specs=[
            pl.BlockSpec((blk, d), lambda i: (i, 0)),
            pl.BlockSpec((blk, d), lambda i: (i, 0)),
            pl.BlockSpec((blk, d), lambda i: (i, 0)),
            pl.BlockSpec(memory_space=pltpu.SMEM),
            wspec, bspec, wspec, bspec,
        ],
        out_specs=[
            pl.BlockSpec((blk, d), lambda i: (i, 0)),
            pl.BlockSpec((blk, d), lambda i: (i, 0)),
        ],
        out_shape=[
            jax.ShapeDtypeStruct((n, d), jnp.float32),
            jax.ShapeDtypeStruct((n, d), jnp.float32),
        ],
    )


@functools.lru_cache(maxsize=None)
def _make_mlp1_pool(n, d, b, blk):
    nb = n // blk

    def body(a0_ref, aggA_ref, aggB_ref, scale_ref, w1_ref, b1_ref,
             w2_ref, b2_ref, h0_ref, batch_ref, lw0_ref, lw1_ref, lb_ref,
             out_ref, p0_ref, p1_ref):
        i = pl.program_id(0)
        h = scale_ref[0, 0] * a0_ref[...] + aggA_ref[...] + aggB_ref[...]
        t = jnp.dot(h, w1_ref[...], preferred_element_type=jnp.float32)
        t = t + b1_ref[...]
        t = jnp.where(t > 0, t, 0.01 * t)
        h1 = jnp.dot(t, w2_ref[...], preferred_element_type=jnp.float32)
        h1 = h1 + b2_ref[...]

        seg = batch_ref[0, 0, :]
        mask = (seg[None, :] ==
                lax.broadcasted_iota(jnp.int32, (b, blk), 0)).astype(
                    jnp.float32)
        pool0 = jnp.dot(mask, h0_ref[...], preferred_element_type=jnp.float32)
        pool1 = jnp.dot(mask, h1, preferred_element_type=jnp.float32)

        @pl.when(i == 0)
        def _():
            p0_ref[...] = jnp.zeros((b, d), jnp.float32)
            p1_ref[...] = jnp.zeros((b, d), jnp.float32)

        p0_ref[...] += pool0
        p1_ref[...] += pool1

        @pl.when(i == nb - 1)
        def _():
            out_ref[...] = (
                jnp.dot(p0_ref[...], lw0_ref[...],
                        preferred_element_type=jnp.float32)
                + jnp.dot(p1_ref[...], lw1_ref[...],
                          preferred_element_type=jnp.float32)
                + lb_ref[...])

    wspec = pl.BlockSpec((d, d), lambda i: (0, 0))
    bspec = pl.BlockSpec((1, d), lambda i: (0, 0))
    return pl.pallas_call(
        body,
        grid=(nb,),
        in_specs=[
            pl.BlockSpec((blk, d), lambda i: (i, 0)),
            pl.BlockSpec((blk, d), lambda i: (i, 0)),
            pl.BlockSpec((blk, d), lambda i: (i, 0)),
            pl.BlockSpec(memory_space=pltpu.SMEM),
            wspec, bspec, wspec, bspec,
            pl.BlockSpec((blk, d), lambda i: (i, 0)),
            pl.BlockSpec((1, 1, blk), lambda i: (i, 0, 0)),
            wspec,
            pl.BlockSpec((d, d), lambda i: (0, 0)),
            bspec,
        ],
        out_specs=pl.BlockSpec((b, d), lambda i: (0, 0)),
        out_shape=jax.ShapeDtypeStruct((b, d), jnp.float32),
        scratch_shapes=[
            pltpu.VMEM((b, d), jnp.float32),
            pltpu.VMEM((b, d), jnp.float32),
        ],
    )


def kernel(x, edge_index, edge_weight, batch, gin0_w1, gin0_b1, gin0_w2,
           gin0_b2, eps0, gin1_w1, gin1_b1, gin1_w2, gin1_b2, eps1,
           lin_w, lin_b):
    n, d = x.shape
    e = edge_weight.shape[0]
    b = 64
    blk = 1000

    nchunks = NS * (T0 + T1)
    e_pad = nchunks * CH
    assert e_pad >= e
    pad = e_pad - e
    src_r = jnp.pad(edge_index[0], (0, pad)).reshape(nchunks, CH)
    dst_r = jnp.pad(edge_index[1], (0, pad)).reshape(nchunks, CH)
    w_r = jnp.pad(edge_weight, (0, pad)).reshape(nchunks, CH)

    sc_agg = _make_sc_aggregate(n, d)
    mlp0 = _make_mlp0(n, d, blk)
    mlp1_pool = _make_mlp1_pool(n, d, b, blk)

    scale0 = (1.0 + eps0).reshape(1, 1).astype(jnp.float32)
    scale1 = (1.0 + eps1).reshape(1, 1).astype(jnp.float32)
    b1_0 = gin0_b1.reshape(1, d)
    b2_0 = gin0_b2.reshape(1, d)
    b1_1 = gin1_b1.reshape(1, d)
    b2_1 = gin1_b2.reshape(1, d)
    lw0 = lin_w[:d]
    lw1 = lin_w[d:]
    lb = lin_b.reshape(1, d)
    batch3 = batch.reshape(n // blk, 1, blk)

    rpt = (((n + NS - 1) // NS + 127) // 128) * 128
    n_pad = NS * rpt

    agg0 = sc_agg(src_r, dst_r, w_r, x)
    h0, a0 = mlp0(x, agg0[:n], agg0[n_pad:n_pad + n], scale0,
                  gin0_w1, b1_0, gin0_w2, b2_0)
    agg1 = sc_agg(src_r, dst_r, w_r, a0)
    return mlp1_pool(a0, agg1[:n], agg1[n_pad:n_pad + n], scale1,
                     gin1_w1, b1_1, gin1_w2, b2_1,
                     h0, batch3, lw0, lw1, lb)


# speedup vs baseline: 1.5524x; 1.1394x over previous
import functools

import jax
import jax.numpy as jnp
from jax import lax
from jax.experimental import pallas as pl
from jax.experimental.pallas import tpu as pltpu
from jax.experimental.pallas import tpu_sc as plsc

NC = 2
NS = 16
NW = NC * NS
CH = 128
SB = 8
LANES = 16
T0 = 144
T1 = 16


@functools.lru_cache(maxsize=None)
def _make_sc_aggregate(n, d):
    rows_per_tile = (((n + NS - 1) // NS + 127) // 128) * 128
    n_pad = NS * rows_per_tile
    zrows = CH
    nzc = rows_per_tile // zrows
    fgroups = d // LANES
    assert T0 % SB == 0 and T1 % SB == 0
    nsb0 = T0 // SB
    nsb1 = T1 // SB

    mesh = plsc.VectorSubcoreMesh(core_axis_name="c", subcore_axis_name="s")

    @functools.partial(
        pl.kernel,
        mesh=mesh,
        out_type=jax.ShapeDtypeStruct((2 * n_pad, d), jnp.float32),
        scratch_types=[
            pltpu.VMEM((SB, CH), jnp.int32),
            pltpu.VMEM((SB, CH), jnp.int32),
            pltpu.VMEM((SB, CH), jnp.float32),
            pltpu.VMEM((CH, d), jnp.float32),
            pltpu.VMEM((CH, d), jnp.float32),
            pltpu.VMEM_SHARED((n_pad, d), jnp.float32),
            pltpu.SemaphoreType.DMA,
            pltpu.SemaphoreType.DMA,
            pltpu.SemaphoreType.DMA,
            pltpu.SemaphoreType.DMA,
        ],
    )
    def agg_kernel(src_hbm, dst_hbm, w_hbm, x_hbm, out_hbm,
                   src_v, dst_v, w_v, rows0, rows1, acc_sh,
                   gs0, gs1, ss0, ss1):
        cid = lax.axis_index("c")
        sid = lax.axis_index("s")
        base_chunk = jnp.where(cid == 0, sid * T0, NS * T0 + sid * T1)
        nsb = jnp.where(cid == 0, nsb0, nsb1)

        def zrow(i, carry):
            for f in range(fgroups):
                rows0[i, pl.ds(f * LANES, LANES)] = jnp.zeros(
                    (LANES,), jnp.float32)
            return carry
        lax.fori_loop(0, zrows, zrow, 0)
        for k in range(nzc):
            pltpu.sync_copy(
                rows0,
                acc_sh.at[pl.ds(sid * rows_per_tile + k * zrows, zrows)])
        plsc.subcore_barrier()

        def scale(buf, j):
            def egroup(g, c2):
                wv16 = w_v[j, pl.ds(g * LANES, LANES)]
                for l in range(LANES):
                    wb = jnp.full((LANES,), wv16[l], dtype=jnp.float32)
                    i = g * LANES + l
                    for f in range(fgroups):
                        sl = pl.ds(f * LANES, LANES)
                        buf[i, sl] = buf[i, sl] * wb
                return c2
            lax.fori_loop(0, CH // LANES, egroup, 0)

        def start_gather(buf, sem, j):
            pltpu.async_copy(x_hbm.at[src_v.at[j]], buf, sem)

        def wait_gather(buf, sem):
            pltpu.make_async_copy(x_hbm.at[src_v.at[0]], buf, sem).wait()

        def start_scatter(buf, sem, j):
            pltpu.async_copy(buf, acc_sh.at[dst_v.at[j]], sem, add=True)

        def wait_scatter(buf, sem):
            pltpu.make_async_copy(buf, acc_sh.at[dst_v.at[0]], sem).wait()

        def block(s, carry):
            cb = base_chunk + s * SB
            pltpu.sync_copy(src_hbm.at[pl.ds(cb, SB)], src_v)
            pltpu.sync_copy(dst_hbm.at[pl.ds(cb, SB)], dst_v)
            pltpu.sync_copy(w_hbm.at[pl.ds(cb, SB)], w_v)

            start_gather(rows0, gs0, 0)
            wait_gather(rows0, gs0)
            scale(rows0, 0)
            start_gather(rows1, gs1, 1)
            start_scatter(rows0, ss0, 0)

            def pair(j2, c1):
                j = 2 * j2 + 1
                wait_gather(rows1, gs1)
                scale(rows1, j)
                wait_scatter(rows0, ss0)
                start_gather(rows0, gs0, j + 1)
                start_scatter(rows1, ss1, j)
                wait_gather(rows0, gs0)
                scale(rows0, j + 1)
                wait_scatter(rows1, ss1)
                start_gather(rows1, gs1, j + 2)
                start_scatter(rows0, ss0, j + 1)
                return c1
            lax.fori_loop(0, (SB - 2) // 2, pair, 0)

            wait_gather(rows1, gs1)
            scale(rows1, SB - 1)
            wait_scatter(rows0, ss0)
            start_scatter(rows1, ss1, SB - 1)
            wait_scatter(rows1, ss1)
            return carry
        lax.fori_loop(0, nsb, block, 0)

        plsc.subcore_barrier()
        pltpu.sync_copy(
            acc_sh.at[pl.ds(sid * rows_per_tile, rows_per_tile)],
            out_hbm.at[pl.ds(cid * n_pad + sid * rows_per_tile,
                             rows_per_tile)])

    return agg_kernel


def _mlp0_body(x_ref, aggA_ref, aggB_ref, scale_ref, w1_ref, b1_ref,
               w2_ref, b2_ref, h0_ref, a0_ref):
    h = scale_ref[0, 0] * x_ref[...] + aggA_ref[...] + aggB_ref[...]
    t = jnp.dot(h, w1_ref[...], preferred_element_type=jnp.float32)
    t = t + b1_ref[...]
    t = jnp.where(t > 0, t, 0.01 * t)
    h0 = jnp.dot(t, w2_ref[...], preferred_element_type=jnp.float32)
    h0 = h0 + b2_ref[...]
    h0_ref[...] = h0
    a0_ref[...] = jnp.where(h0 > 0, h0, 0.01 * h0)


@functools.lru_cache(maxsize=None)
def _make_mlp0(n, d, blk):
    nb = n // blk
    wspec = pl.BlockSpec((d, d), lambda i: (0, 0))
    bspec = pl.BlockSpec((1, d), lambda i: (0, 0))
    return pl.pallas_call(
        _mlp0_body,
        grid=(nb,),
        in_specs=[
            pl.BlockSpec((blk, d), lambda i: (i, 0)),
            pl.BlockSpec((blk, d), lambda i: (i, 0)),
            pl.BlockSpec((blk, d), lambda i: (i, 0)),
            pl.BlockSpec(memory_space=pltpu.SMEM),
            wspec, bspec, wspec, bspec,
        ],
        out_specs=[
            pl.BlockSpec((blk, d), lambda i: (i, 0)),
            pl.BlockSpec((blk, d), lambda i: (i, 0)),
        ],
        out_shape=[
            jax.ShapeDtypeStruct((n, d), jnp.float32),
            jax.ShapeDtypeStruct((n, d), jnp.float32),
        ],
    )


@functools.lru_cache(maxsize=None)
def _make_mlp1_pool(n, d, b, blk):
    nb = n // blk

    def body(a0_ref, aggA_ref, aggB_ref, scale_ref, w1_ref, b1_ref,
             w2_ref, b2_ref, h0_ref, batch_ref, lw0_ref, lw1_ref, lb_ref,
             out_ref, p0_ref, p1_ref):
        i = pl.program_id(0)
        h = scale_ref[0, 0] * a0_ref[...] + aggA_ref[...] + aggB_ref[...]
        t = jnp.dot(h, w1_ref[...], preferred_element_type=jnp.float32)
        t = t + b1_ref[...]
        t = jnp.where(t > 0, t, 0.01 * t)
        h1 = jnp.dot(t, w2_ref[...], preferred_element_type=jnp.float32)
        h1 = h1 + b2_ref[...]

        seg = batch_ref[0, 0, :]
        mask = (seg[None, :] ==
                lax.broadcasted_iota(jnp.int32, (b, blk), 0)).astype(
                    jnp.float32)
        pool0 = jnp.dot(mask, h0_ref[...], preferred_element_type=jnp.float32)
        pool1 = jnp.dot(mask, h1, preferred_element_type=jnp.float32)

        @pl.when(i == 0)
        def _():
            p0_ref[...] = jnp.zeros((b, d), jnp.float32)
            p1_ref[...] = jnp.zeros((b, d), jnp.float32)

        p0_ref[...] += pool0
        p1_ref[...] += pool1

        @pl.when(i == nb - 1)
        def _():
            out_ref[...] = (
                jnp.dot(p0_ref[...], lw0_ref[...],
                        preferred_element_type=jnp.float32)
                + jnp.dot(p1_ref[...], lw1_ref[...],
                          preferred_element_type=jnp.float32)
                + lb_ref[...])

    wspec = pl.BlockSpec((d, d), lambda i: (0, 0))
    bspec = pl.BlockSpec((1, d), lambda i: (0, 0))
    return pl.pallas_call(
        body,
        grid=(nb,),
        in_specs=[
            pl.BlockSpec((blk, d), lambda i: (i, 0)),
            pl.BlockSpec((blk, d), lambda i: (i, 0)),
            pl.BlockSpec((blk, d), lambda i: (i, 0)),
            pl.BlockSpec(memory_space=pltpu.SMEM),
            wspec, bspec, wspec, bspec,
            pl.BlockSpec((blk, d), lambda i: (i, 0)),
            pl.BlockSpec((1, 1, blk), lambda i: (i, 0, 0)),
            wspec,
            pl.BlockSpec((d, d), lambda i: (0, 0)),
            bspec,
        ],
        out_specs=pl.BlockSpec((b, d), lambda i: (0, 0)),
        out_shape=jax.ShapeDtypeStruct((b, d), jnp.float32),
        scratch_shapes=[
            pltpu.VMEM((b, d), jnp.float32),
            pltpu.VMEM((b, d), jnp.float32),
        ],
    )


def kernel(x, edge_index, edge_weight, batch, gin0_w1, gin0_b1, gin0_w2,
           gin0_b2, eps0, gin1_w1, gin1_b1, gin1_w2, gin1_b2, eps1,
           lin_w, lin_b):
    n, d = x.shape
    e = edge_weight.shape[0]
    b = 64
    blk = 1000

    nchunks = NS * (T0 + T1)
    e_pad = nchunks * CH
    assert e_pad >= e
    pad = e_pad - e
    src_r = jnp.pad(edge_index[0], (0, pad)).reshape(nchunks, CH)
    dst_r = jnp.pad(edge_index[1], (0, pad)).reshape(nchunks, CH)
    w_r = jnp.pad(edge_weight, (0, pad)).reshape(nchunks, CH)

    sc_agg = _make_sc_aggregate(n, d)
    mlp0 = _make_mlp0(n, d, blk)
    mlp1_pool = _make_mlp1_pool(n, d, b, blk)

    scale0 = (1.0 + eps0).reshape(1, 1).astype(jnp.float32)
    scale1 = (1.0 + eps1).reshape(1, 1).astype(jnp.float32)
    b1_0 = gin0_b1.reshape(1, d)
    b2_0 = gin0_b2.reshape(1, d)
    b1_1 = gin1_b1.reshape(1, d)
    b2_1 = gin1_b2.reshape(1, d)
    lw0 = lin_w[:d]
    lw1 = lin_w[d:]
    lb = lin_b.reshape(1, d)
    batch3 = batch.reshape(n // blk, 1, blk)

    rpt = (((n + NS - 1) // NS + 127) // 128) * 128
    n_pad = NS * rpt

    agg0 = sc_agg(src_r, dst_r, w_r, x)
    h0, a0 = mlp0(x, agg0[:n], agg0[n_pad:n_pad + n], scale0,
                  gin0_w1, b1_0, gin0_w2, b2_0)
    agg1 = sc_agg(src_r, dst_r, w_r, a0)
    return mlp1_pool(a0, agg1[:n], agg1[n_pad:n_pad + n], scale1,
                     gin1_w1, b1_1, gin1_w2, b2_1,
                     h0, batch3, lw0, lw1, lb)
